# baseline (speedup 1.0000x reference)
import functools

import jax
import jax.numpy as jnp
from jax import lax
from jax.experimental import pallas as pl
from jax.experimental.pallas import tpu as pltpu
from jax.experimental.pallas import tpu_sc as plsc

BATCH = 16384
EMB_K = 64
CHUNK = 128


def _sc_info():
  info = plsc.get_sparse_core_info()
  return info.num_cores, info.num_subcores, info.num_lanes


def _gather_rows_call(idx, table):
  nc, ns, nl = _sc_info()
  nw = nc * ns
  rows_per_w = BATCH // nw
  half = rows_per_w // 2
  groups_per_half = half // nl

  mesh = plsc.VectorSubcoreMesh(core_axis_name="c", subcore_axis_name="s")

  @functools.partial(
      pl.kernel,
      mesh=mesh,
      out_type=jax.ShapeDtypeStruct((BATCH, EMB_K), jnp.float32),
      scratch_types=[
          pltpu.VMEM((rows_per_w,), jnp.int32),
          pltpu.VMEM((half, EMB_K), jnp.float32),
          pltpu.SemaphoreType.DMA,
      ],
  )
  def gather_k(idx_hbm, w_hbm, u_out, idx_v, rows, sem):
    wid = lax.axis_index("s") * nc + lax.axis_index("c")
    row_base = wid * rows_per_w
    pltpu.sync_copy(idx_hbm.at[pl.ds(row_base, rows_per_w)], idx_v)

    for h in range(2):
      def group(g, _, h=h):
        vec = idx_v[pl.ds(h * half + g * nl, nl)]
        cps = []
        for l in range(nl):
          cps.append(pltpu.async_copy(
              w_hbm.at[pl.ds(vec[l], 1)],
              rows.at[pl.ds(g * nl + l, 1)], sem))
        for cp in cps:
          cp.wait()
        return 0

      lax.fori_loop(0, groups_per_half, group, 0)
      pltpu.sync_copy(rows, u_out.at[pl.ds(row_base + h * half, half)])

  return gather_k(idx, table)


def _gather_stream_call(idx2d, table):
  nc, ns, nl = _sc_info()
  nw = nc * ns
  rows_per_w = BATCH // nw
  chunks_per_w = rows_per_w // CHUNK

  mesh = plsc.VectorSubcoreMesh(core_axis_name="c", subcore_axis_name="s")

  @functools.partial(
      pl.kernel,
      mesh=mesh,
      out_type=jax.ShapeDtypeStruct((BATCH, EMB_K), jnp.float32),
      scratch_types=[
          pltpu.VMEM((chunks_per_w, CHUNK), jnp.int32),
          pltpu.VMEM((rows_per_w, EMB_K), jnp.float32),
          pltpu.SemaphoreType.DMA,
      ],
      compiler_params=pltpu.CompilerParams(use_tc_tiling_on_sc=False),
  )
  def gather_k(idx_hbm, h_hbm, v_out, idx_v, rows, sem):
    wid = lax.axis_index("s") * nc + lax.axis_index("c")
    idx_base = wid * chunks_per_w
    pltpu.sync_copy(idx_hbm.at[pl.ds(idx_base, chunks_per_w)], idx_v)
    cps = []
    for c in range(chunks_per_w):
      cps.append(pltpu.async_copy(
          h_hbm.at[idx_v.at[c]], rows.at[pl.ds(c * CHUNK, CHUNK)], sem))
    for cp in cps:
      cp.wait()
    row_base = wid * rows_per_w
    pltpu.sync_copy(rows, v_out.at[pl.ds(row_base, rows_per_w)])

  return gather_k(idx2d, table)


def _mlp_body(u_ref, v_ref, w1a_ref, w1b_ref, b1_ref, w2_ref, b2_ref,
              w3_ref, b3_ref, out_ref):
  u = u_ref[...]
  v = v_ref[...]
  h = jnp.dot(u, w1a_ref[...], preferred_element_type=jnp.float32)
  h += jnp.dot(v, w1b_ref[...], preferred_element_type=jnp.float32)
  h = jnp.maximum(h + b1_ref[...], 0.0)
  h = jnp.dot(h, w2_ref[...], preferred_element_type=jnp.float32)
  h = jnp.maximum(h + b2_ref[...], 0.0)
  out_ref[...] = jnp.sum(h * w3_ref[...], axis=1) + b3_ref[0]


def _mlp_call(U, V, W1aT, W1bT, b1, W2T, b2, w3, b3):
  blk = 2048
  grid = (BATCH // blk,)
  full = lambda shape: pl.BlockSpec(shape, lambda i: (0,) * len(shape))
  return pl.pallas_call(
      _mlp_body,
      grid=grid,
      in_specs=[
          pl.BlockSpec((blk, EMB_K), lambda i: (i, 0)),
          pl.BlockSpec((blk, EMB_K), lambda i: (i, 0)),
          full((EMB_K, EMB_K)),
          full((EMB_K, EMB_K)),
          full((1, EMB_K)),
          full((EMB_K, EMB_K)),
          full((1, EMB_K)),
          full((1, EMB_K)),
          full((1,)),
      ],
      out_specs=pl.BlockSpec((blk,), lambda i: (i,)),
      out_shape=jax.ShapeDtypeStruct((BATCH,), jnp.float32),
  )(U, V, W1aT, W1bT, b1, W2T, b2, w3, b3)


@jax.jit
def kernel(x, W, H, W1, b1, W2, b2, W3, b3):
  uidx = x[:, 0].astype(jnp.int32)
  vidx = x[:, 1].astype(jnp.int32).reshape(BATCH // CHUNK, CHUNK)
  V = _gather_stream_call(vidx, H)
  U = _gather_rows_call(uidx, W)
  out = _mlp_call(
      U, V,
      W1[:, :EMB_K].T, W1[:, EMB_K:].T, b1.reshape(1, EMB_K),
      W2.T, b2.reshape(1, EMB_K),
      W3.reshape(1, EMB_K), b3,
  )
  return out


# speedup vs baseline: 1.4393x; 1.4393x over previous
import functools

import jax
import jax.numpy as jnp
from jax import lax
from jax.experimental import pallas as pl
from jax.experimental.pallas import tpu as pltpu
from jax.experimental.pallas import tpu_sc as plsc

BATCH = 16384
EMB_K = 64
PACKED = 2 * EMB_K
CHUNK = 128
PACK_BLK = 2048


def _pack_body(wt_ref, ht_ref, p_ref):
  p_ref[...] = jnp.concatenate([wt_ref[...].T, ht_ref[...].T], axis=1)


def _pack_call(WT, HT):
  n = WT.shape[1]
  grid = (pl.cdiv(n, PACK_BLK),)
  return pl.pallas_call(
      _pack_body,
      grid=grid,
      in_specs=[
          pl.BlockSpec((EMB_K, PACK_BLK), lambda i: (0, i)),
          pl.BlockSpec((EMB_K, PACK_BLK), lambda i: (0, i)),
      ],
      out_specs=pl.BlockSpec((PACK_BLK, PACKED), lambda i: (i, 0)),
      out_shape=jax.ShapeDtypeStruct((n, PACKED), jnp.float32),
  )(WT, HT)


def _gather_call(uidx3d, vidx3d, P):
  info = plsc.get_sparse_core_info()
  nc, ns = info.num_cores, info.num_subcores
  nw = nc * ns
  rows_per_w = BATCH // nw
  chunks_per_w = rows_per_w // CHUNK

  mesh = plsc.VectorSubcoreMesh(core_axis_name="c", subcore_axis_name="s")

  @functools.partial(
      pl.kernel,
      mesh=mesh,
      out_type=[
          jax.ShapeDtypeStruct((BATCH, PACKED), jnp.float32),
          jax.ShapeDtypeStruct((BATCH, PACKED), jnp.float32),
      ],
      scratch_types=[
          pltpu.VMEM((chunks_per_w, 1, CHUNK), jnp.int32),
          pltpu.VMEM((chunks_per_w, 1, CHUNK), jnp.int32),
          pltpu.VMEM((rows_per_w, PACKED), jnp.float32),
          pltpu.SemaphoreType.DMA,
      ],
  )
  def gather_k(uidx_hbm, vidx_hbm, p_hbm, u_out, v_out,
               uidx_v, vidx_v, rows, sem):
    wid = lax.axis_index("s") * nc + lax.axis_index("c")
    idx_base = wid * chunks_per_w
    row_base = wid * rows_per_w
    pltpu.sync_copy(uidx_hbm.at[pl.ds(idx_base, chunks_per_w)], uidx_v)
    pltpu.sync_copy(vidx_hbm.at[pl.ds(idx_base, chunks_per_w)], vidx_v)
    for idx_v, out in ((uidx_v, u_out), (vidx_v, v_out)):
      cps = []
      for c in range(chunks_per_w):
        cps.append(pltpu.async_copy(
            p_hbm.at[idx_v.at[c, 0]], rows.at[pl.ds(c * CHUNK, CHUNK)], sem))
      for cp in cps:
        cp.wait()
      pltpu.sync_copy(rows, out.at[pl.ds(row_base, rows_per_w)])

  return gather_k(uidx3d, vidx3d, P)


def _mlp_body(u_ref, v_ref, p1u_ref, p1v_ref, b1_ref, w2_ref, b2_ref,
              w3_ref, b3_ref, out_ref):
  h = jnp.dot(u_ref[...], p1u_ref[...], preferred_element_type=jnp.float32)
  h += jnp.dot(v_ref[...], p1v_ref[...], preferred_element_type=jnp.float32)
  h = jnp.maximum(h + b1_ref[...], 0.0)
  h = jnp.dot(h, w2_ref[...], preferred_element_type=jnp.float32)
  h = jnp.maximum(h + b2_ref[...], 0.0)
  out_ref[...] = jnp.sum(h * w3_ref[...], axis=1) + b3_ref[0]


def _mlp_call(U128, V128, P1u, P1v, b1, W2T, b2, w3, b3):
  blk = 2048
  grid = (BATCH // blk,)
  full = lambda shape: pl.BlockSpec(shape, lambda i: (0,) * len(shape))
  return pl.pallas_call(
      _mlp_body,
      grid=grid,
      in_specs=[
          pl.BlockSpec((blk, PACKED), lambda i: (i, 0)),
          pl.BlockSpec((blk, PACKED), lambda i: (i, 0)),
          full((PACKED, EMB_K)),
          full((PACKED, EMB_K)),
          full((1, EMB_K)),
          full((EMB_K, EMB_K)),
          full((1, EMB_K)),
          full((1, EMB_K)),
          full((1,)),
      ],
      out_specs=pl.BlockSpec((blk,), lambda i: (i,)),
      out_shape=jax.ShapeDtypeStruct((BATCH,), jnp.float32),
  )(U128, V128, P1u, P1v, b1, W2T, b2, w3, b3)


@jax.jit
def kernel(x, W, H, W1, b1, W2, b2, W3, b3):
  uidx = x[:, 0].astype(jnp.int32).reshape(BATCH // CHUNK, 1, CHUNK)
  vidx = x[:, 1].astype(jnp.int32).reshape(BATCH // CHUNK, 1, CHUNK)
  P = _pack_call(W.T, H.T)
  U128, V128 = _gather_call(uidx, vidx, P)
  w1t = W1.T
  zeros = jnp.zeros((EMB_K, EMB_K), jnp.float32)
  P1u = jnp.concatenate([w1t[:EMB_K], zeros], axis=0)
  P1v = jnp.concatenate([zeros, w1t[EMB_K:]], axis=0)
  out = _mlp_call(
      U128, V128, P1u, P1v, b1.reshape(1, EMB_K),
      W2.T, b2.reshape(1, EMB_K),
      W3.reshape(1, EMB_K), b3,
  )
  return out
